# baseline (speedup 1.0000x reference)
import jax
import jax.numpy as jnp
from jax import lax
from jax.experimental import pallas as pl
from jax.experimental.pallas import tpu as pltpu

B, V, K = 128, 100000, 10
BT = 16
G = 896
CH = 128
NFULL = V // G
TAIL = V - NFULL * G
SENT = -1e30
BIGI = 1 << 30


def _body(x_ref, tok_ref, w1_ref, b1_ref, w2_ref, b2_ref, w3_ref, b3_ref,
          out_ref, x3_ref, f3_ref, cand_ref):
    for m in range(NFULL):
        x3_ref[:, m, :] = x_ref[:, m * G:(m + 1) * G]
    x3_ref[:, NFULL, :TAIL] = x_ref[:, NFULL * G:]
    x3_ref[:, NFULL, TAIL:] = jnp.full((BT, G - TAIL), SENT, jnp.float32)
    x3_ref[:, NFULL + 1:, :] = jnp.full((BT, CH - NFULL - 1, G), SENT,
                                        jnp.float32)
    x3 = x3_ref[...]

    gmax = jnp.max(x3, axis=1)
    giota = lax.broadcasted_iota(jnp.int32, (BT, G), 1)
    for k in range(K):
        mk = jnp.max(gmax, axis=1, keepdims=True)
        sel = jnp.where(gmax == mk, giota, BIGI)
        idx = jnp.min(sel, axis=1, keepdims=True)
        fk = giota == idx
        f3_ref[:, k, :] = fk.astype(jnp.float32)
        gmax = jnp.where(fk, SENT, gmax)
    tok = tok_ref[...]
    gt = tok % G
    f3_ref[:, K, :] = (giota == gt).astype(jnp.float32)

    for b in range(BT):
        cb = lax.dot_general(x3_ref[b], f3_ref[b], (((1,), (1,)), ((), ())),
                             precision=lax.Precision.HIGHEST)
        cand_ref[b] = cb

    mt = tok // G
    chiota = lax.broadcasted_iota(jnp.int32, (BT, CH), 1)
    ctok = cand_ref[:, :, K]
    xt = jnp.sum(jnp.where(chiota == mt, ctok, 0.0), axis=1, keepdims=True)

    kiota = lax.broadcasted_iota(jnp.int32, (BT, CH, 16), 2)
    cand = jnp.where(kiota < K, cand_ref[...], SENT)
    ms = []
    cs = []
    for _ in range(K):
        m = jnp.max(cand, axis=(1, 2), keepdims=True)
        eq = cand == m
        c = jnp.sum(eq.astype(jnp.float32), axis=(1, 2), keepdims=True)
        cand = jnp.where(eq, SENT, cand)
        ms.append(m[:, :, 0])
        cs.append(c[:, :, 0])

    j16 = lax.broadcasted_iota(jnp.int32, (BT, 16), 1).astype(jnp.float32)
    t = jnp.zeros((BT, 16), jnp.float32)
    cum = jnp.zeros((BT, 1), jnp.float32)
    for m, c in zip(ms, cs):
        nxt = cum + c
        t = jnp.where((j16 >= cum) & (j16 < nxt), m, t)
        cum = nxt

    def r16(v):
        return v.astype(jnp.bfloat16).astype(jnp.float32)

    tb = r16(t)
    h1 = []
    for j in range(5):
        acc = tb[:, 0:1] * r16(w1_ref[j, 0])
        for i in range(1, 10):
            acc = acc + tb[:, i:i + 1] * r16(w1_ref[j, i])
        h1.append(jnp.maximum(acc + b1_ref[j], 0.0))
    h1 = [r16(v) for v in h1]
    h2 = []
    for j in range(5):
        acc = h1[0] * r16(w2_ref[j, 0])
        for i in range(1, 5):
            acc = acc + h1[i] * r16(w2_ref[j, i])
        h2.append(jnp.maximum(acc + b2_ref[j], 0.0))
    h2 = [r16(v) for v in h2]
    h3 = h2[0] * r16(w3_ref[0, 0])
    for i in range(1, 5):
        h3 = h3 + h2[i] * r16(w3_ref[0, i])
    z = jnp.abs(h3 + b3_ref[0])
    temp = z + jnp.log1p(jnp.exp(-z))
    temp = jnp.maximum(temp, jnp.float32(1e-5))
    inv_t = 1.0 / temp

    m0 = ms[0][:, :, None]
    ex = jnp.exp((x3 - m0) * inv_t[:, :, None])
    s = jnp.sum(ex, axis=(1, 2), keepdims=True)[:, :, 0]
    out_ref[...] = jnp.exp((xt - ms[0]) * inv_t) / s


@jax.jit
def kernel(inp, tokens, W1, b1, W2, b2, W3, b3):
    grid = (B // BT,)
    out = pl.pallas_call(
        _body,
        grid=grid,
        in_specs=[
            pl.BlockSpec((BT, V), lambda i: (i, 0)),
            pl.BlockSpec((BT, 1), lambda i: (i, 0)),
            pl.BlockSpec(memory_space=pltpu.SMEM),
            pl.BlockSpec(memory_space=pltpu.SMEM),
            pl.BlockSpec(memory_space=pltpu.SMEM),
            pl.BlockSpec(memory_space=pltpu.SMEM),
            pl.BlockSpec(memory_space=pltpu.SMEM),
            pl.BlockSpec(memory_space=pltpu.SMEM),
        ],
        out_specs=pl.BlockSpec((BT, 1), lambda i: (i, 0)),
        out_shape=jax.ShapeDtypeStruct((B, 1), jnp.float32),
        scratch_shapes=[pltpu.VMEM((BT, CH, G), jnp.float32),
                        pltpu.VMEM((BT, 16, G), jnp.float32),
                        pltpu.VMEM((BT, CH, 16), jnp.float32)],
    )(inp, tokens[:, None], W1, b1, W2, b2, W3, b3)
    return out[:, 0]


# speedup vs baseline: 1.0584x; 1.0584x over previous
import jax
import jax.numpy as jnp
from jax import lax
from jax.experimental import pallas as pl
from jax.experimental.pallas import tpu as pltpu

B, V, K = 128, 100000, 10
BT = 32
G = 896
NFULL = V // G
CH = NFULL + 1
TAIL = V - NFULL * G
SENT = -1e30
BIGI = 1 << 30


def _body(x_ref, tok_ref, w1_ref, b1_ref, w2_ref, b2_ref, w3_ref, b3_ref,
          out_ref, x3_ref, f3_ref, cand_ref):
    for m in range(NFULL):
        x3_ref[:, m, :] = x_ref[:, m * G:(m + 1) * G]
    x3_ref[:, NFULL, :TAIL] = x_ref[:, NFULL * G:]
    x3_ref[:, NFULL, TAIL:] = jnp.full((BT, G - TAIL), SENT, jnp.float32)
    x3 = x3_ref[...]

    gmax = jnp.max(x3, axis=1)
    giota = lax.broadcasted_iota(jnp.int32, (BT, G), 1)
    for k in range(K):
        mk = jnp.max(gmax, axis=1, keepdims=True)
        sel = jnp.where(gmax == mk, giota, BIGI)
        idx = jnp.min(sel, axis=1, keepdims=True)
        fk = giota == idx
        f3_ref[:, k, :] = fk.astype(jnp.float32)
        gmax = jnp.where(fk, SENT, gmax)
    tok = tok_ref[...]
    gt = tok % G
    f3_ref[:, K, :] = (giota == gt).astype(jnp.float32)

    for b in range(BT):
        cb = lax.dot_general(x3_ref[b], f3_ref[b], (((1,), (1,)), ((), ())),
                             precision=lax.Precision.HIGHEST)
        cand_ref[b] = cb

    mt = tok // G
    chiota = lax.broadcasted_iota(jnp.int32, (BT, CH), 1)
    ctok = cand_ref[:, :, K]
    xt = jnp.sum(jnp.where(chiota == mt, ctok, 0.0), axis=1, keepdims=True)

    kiota = lax.broadcasted_iota(jnp.int32, (BT, CH, 16), 2)
    cand = jnp.where(kiota < K, cand_ref[...], SENT)
    ms = []
    cs = []
    for _ in range(K):
        m = jnp.max(cand, axis=(1, 2), keepdims=True)
        eq = cand == m
        c = jnp.sum(eq.astype(jnp.float32), axis=(1, 2), keepdims=True)
        cand = jnp.where(eq, SENT, cand)
        ms.append(m[:, :, 0])
        cs.append(c[:, :, 0])

    j16 = lax.broadcasted_iota(jnp.int32, (BT, 16), 1).astype(jnp.float32)
    t = jnp.zeros((BT, 16), jnp.float32)
    cum = jnp.zeros((BT, 1), jnp.float32)
    for m, c in zip(ms, cs):
        nxt = cum + c
        t = jnp.where((j16 >= cum) & (j16 < nxt), m, t)
        cum = nxt

    def r16(v):
        return v.astype(jnp.bfloat16).astype(jnp.float32)

    tb = r16(t)
    h1 = []
    for j in range(5):
        acc = tb[:, 0:1] * r16(w1_ref[j, 0])
        for i in range(1, 10):
            acc = acc + tb[:, i:i + 1] * r16(w1_ref[j, i])
        h1.append(jnp.maximum(acc + b1_ref[j], 0.0))
    h1 = [r16(v) for v in h1]
    h2 = []
    for j in range(5):
        acc = h1[0] * r16(w2_ref[j, 0])
        for i in range(1, 5):
            acc = acc + h1[i] * r16(w2_ref[j, i])
        h2.append(jnp.maximum(acc + b2_ref[j], 0.0))
    h2 = [r16(v) for v in h2]
    h3 = h2[0] * r16(w3_ref[0, 0])
    for i in range(1, 5):
        h3 = h3 + h2[i] * r16(w3_ref[0, i])
    z = jnp.abs(h3 + b3_ref[0])
    temp = z + jnp.log1p(jnp.exp(-z))
    temp = jnp.maximum(temp, jnp.float32(1e-5))
    inv_t = 1.0 / temp

    m0 = ms[0][:, :, None]
    ex = jnp.exp((x3 - m0) * inv_t[:, :, None])
    s = jnp.sum(ex, axis=(1, 2), keepdims=True)[:, :, 0]
    out_ref[...] = jnp.exp((xt - ms[0]) * inv_t) / s


@jax.jit
def kernel(inp, tokens, W1, b1, W2, b2, W3, b3):
    grid = (B // BT,)
    out = pl.pallas_call(
        _body,
        grid=grid,
        in_specs=[
            pl.BlockSpec((BT, V), lambda i: (i, 0)),
            pl.BlockSpec((BT, 1), lambda i: (i, 0)),
            pl.BlockSpec(memory_space=pltpu.SMEM),
            pl.BlockSpec(memory_space=pltpu.SMEM),
            pl.BlockSpec(memory_space=pltpu.SMEM),
            pl.BlockSpec(memory_space=pltpu.SMEM),
            pl.BlockSpec(memory_space=pltpu.SMEM),
            pl.BlockSpec(memory_space=pltpu.SMEM),
        ],
        out_specs=pl.BlockSpec((BT, 1), lambda i: (i, 0)),
        out_shape=jax.ShapeDtypeStruct((B, 1), jnp.float32),
        scratch_shapes=[pltpu.VMEM((BT, CH, G), jnp.float32),
                        pltpu.VMEM((BT, 16, G), jnp.float32),
                        pltpu.VMEM((BT, CH, 16), jnp.float32)],
    )(inp, tokens[:, None], W1, b1, W2, b2, W3, b3)
    return out[:, 0]
